# baseline (speedup 1.0000x reference)
import functools

import jax
import jax.numpy as jnp
from jax import lax
from jax.experimental import pallas as pl
from jax.experimental.pallas import tpu as pltpu
from jax.experimental.pallas import tpu_sc as plsc

_B = 16384
_D = 128

_NC = 2
_NS = 16
_NW = _NC * _NS
_ROWS_PER_W = _B // _NW
_CH = 256


@functools.cache
def _make_gather():
    mesh = plsc.VectorSubcoreMesh(core_axis_name="c", subcore_axis_name="s")

    @functools.partial(
        pl.kernel,
        mesh=mesh,
        out_type=[
            jax.ShapeDtypeStruct((_B, _D), jnp.float32),
            jax.ShapeDtypeStruct((_B, _D), jnp.float32),
        ],
        scratch_types=[
            pltpu.VMEM((_ROWS_PER_W,), jnp.int32),
            pltpu.VMEM((_ROWS_PER_W,), jnp.int32),
            pltpu.VMEM((_CH, _D), jnp.float32),
            pltpu.VMEM((_CH, _D), jnp.float32),
            pltpu.SemaphoreType.DMA,
            pltpu.SemaphoreType.DMA,
        ],
    )
    def _gather2(uidx_hbm, iidx_hbm, table_hbm, out_x, out_y,
                 uix_v, iix_v, bufa, bufb, sema, semb):
        wid = lax.axis_index("s") * _NC + lax.axis_index("c")
        base = wid * _ROWS_PER_W
        pltpu.sync_copy(uidx_hbm.at[pl.ds(base, _ROWS_PER_W)], uix_v)
        pltpu.sync_copy(iidx_hbm.at[pl.ds(base, _ROWS_PER_W)], iix_v)
        sched = [
            (uix_v, 0, out_x),
            (uix_v, _CH, out_x),
            (iix_v, 0, out_y),
            (iix_v, _CH, out_y),
        ]
        bufs = [(bufa, sema), (bufb, semb)]
        copies = []
        for k, (idx_v, off, _out) in enumerate(sched):
            buf, sem = bufs[k % 2]
            copies.append(
                pltpu.async_copy(table_hbm.at[idx_v.at[pl.ds(off, _CH)]], buf, sem)
            )
            if k >= 1:
                pidx = k - 1
                copies[pidx].wait()
                pbuf, _ = bufs[pidx % 2]
                _, poff, pout = sched[pidx]
                pltpu.sync_copy(pbuf, pout.at[pl.ds(base + poff, _CH)])
        copies[-1].wait()
        _, loff, lout = sched[-1]
        pltpu.sync_copy(bufs[3 % 2][0], lout.at[pl.ds(base + loff, _CH)])

    return _gather2


_BS = 2048


def _mlp_body(x_ref, y_ref, w1a, w1b, b1, w2, b2, w3, b3, w4r, b4, o_ref):
    h = jnp.dot(x_ref[...], w1a[...], preferred_element_type=jnp.float32)
    h = h + jnp.dot(y_ref[...], w1b[...], preferred_element_type=jnp.float32)
    h = jnp.maximum(h + b1[...], 0.0)
    h = jnp.maximum(jnp.dot(h, w2[...], preferred_element_type=jnp.float32) + b2[...], 0.0)
    h = jnp.maximum(jnp.dot(h, w3[...], preferred_element_type=jnp.float32) + b3[...], 0.0)
    z = jnp.dot(h, w4r[...], preferred_element_type=jnp.float32)[:, 0] + b4[0]
    o_ref[...] = 1.0 / (1.0 + jnp.exp(-z))


def _full(shape):
    return pl.BlockSpec(shape, lambda i: tuple(0 for _ in shape))


def _mlp(xg, yg, w1a, w1b, b1, w2, b2, w3, b3, w4r, b4):
    return pl.pallas_call(
        _mlp_body,
        grid=(_B // _BS,),
        in_specs=[
            pl.BlockSpec((_BS, _D), lambda i: (i, 0)),
            pl.BlockSpec((_BS, _D), lambda i: (i, 0)),
            _full((_D, 128)),
            _full((_D, 128)),
            _full((1, 128)),
            _full((128, 64)),
            _full((1, 64)),
            _full((64, 32)),
            _full((1, 32)),
            _full((32, 1)),
            _full((1,)),
        ],
        out_specs=pl.BlockSpec((_BS,), lambda i: (i,)),
        out_shape=jax.ShapeDtypeStruct((_B,), jnp.float32),
        compiler_params=pltpu.CompilerParams(dimension_semantics=("parallel",)),
    )(xg, yg, w1a, w1b, b1, w2, b2, w3, b3, w4r, b4)


def kernel(user_input, item_input, user_emb, W1, b1, W2, b2, W3, b3, W4, b4):
    uidx = user_input.astype(jnp.int32)
    iidx = item_input.astype(jnp.int32)
    xg, yg = _make_gather()(uidx, iidx, user_emb)
    return _mlp(
        xg, yg,
        W1[:_D], W1[_D:],
        b1.reshape(1, -1),
        W2, b2.reshape(1, -1),
        W3, b3.reshape(1, -1),
        W4, b4,
    )


# speedup vs baseline: 1.0347x; 1.0347x over previous
import functools

import jax
import jax.numpy as jnp
from jax import lax
from jax.experimental import pallas as pl
from jax.experimental.pallas import tpu as pltpu
from jax.experimental.pallas import tpu_sc as plsc

_B = 16384
_D = 128

_NC = 2
_NS = 16
_NW = _NC * _NS
_ROWS_PER_W = _B // _NW
_CH = 256


@functools.cache
def _make_gather():
    mesh = plsc.VectorSubcoreMesh(core_axis_name="c", subcore_axis_name="s")

    @functools.partial(
        pl.kernel,
        mesh=mesh,
        out_type=[
            jax.ShapeDtypeStruct((_B, _D), jnp.float32),
            jax.ShapeDtypeStruct((_B, _D), jnp.float32),
        ],
        scratch_types=[
            pltpu.VMEM((_ROWS_PER_W,), jnp.int32),
            pltpu.VMEM((_ROWS_PER_W,), jnp.int32),
            pltpu.VMEM((_CH, _D), jnp.float32),
            pltpu.VMEM((_CH, _D), jnp.float32),
            pltpu.VMEM((_CH, _D), jnp.float32),
            pltpu.SemaphoreType.DMA,
            pltpu.SemaphoreType.DMA,
            pltpu.SemaphoreType.DMA,
            pltpu.SemaphoreType.DMA,
        ],
    )
    def _gather2(uidx_hbm, iidx_hbm, table_hbm, out_x, out_y,
                 uix_v, iix_v, bufa, bufb, bufc, gsa, gsb, gsc, st_sem):
        wid = lax.axis_index("s") * _NC + lax.axis_index("c")
        base = wid * _ROWS_PER_W
        pltpu.sync_copy(uidx_hbm.at[pl.ds(base, _ROWS_PER_W)], uix_v)
        pltpu.sync_copy(iidx_hbm.at[pl.ds(base, _ROWS_PER_W)], iix_v)
        sched = [
            (uix_v, 0, out_x),
            (uix_v, _CH, out_x),
            (iix_v, 0, out_y),
            (iix_v, _CH, out_y),
        ]
        bufs = [(bufa, gsa), (bufb, gsb), (bufc, gsc)]
        gathers = []
        stores = []
        for k, (idx_v, off, _out) in enumerate(sched):
            buf, sem = bufs[k % 3]
            gathers.append(
                pltpu.async_copy(table_hbm.at[idx_v.at[pl.ds(off, _CH)]], buf, sem)
            )
            if k >= 2:
                pidx = k - 2
                gathers[pidx].wait()
                pbuf, _ = bufs[pidx % 3]
                _, poff, pout = sched[pidx]
                stores.append(
                    pltpu.async_copy(pbuf, pout.at[pl.ds(base + poff, _CH)], st_sem)
                )
        for pidx in (len(sched) - 2, len(sched) - 1):
            gathers[pidx].wait()
            pbuf, _ = bufs[pidx % 3]
            _, poff, pout = sched[pidx]
            stores.append(
                pltpu.async_copy(pbuf, pout.at[pl.ds(base + poff, _CH)], st_sem)
            )
        for st in stores:
            st.wait()

    return _gather2


_BS = 2048


def _mlp_body(x_ref, y_ref, w1a, w1b, b1, w2, b2, w3, b3, w4r, b4, o_ref):
    h = jnp.dot(x_ref[...], w1a[...], preferred_element_type=jnp.float32)
    h = h + jnp.dot(y_ref[...], w1b[...], preferred_element_type=jnp.float32)
    h = jnp.maximum(h + b1[...], 0.0)
    h = jnp.maximum(jnp.dot(h, w2[...], preferred_element_type=jnp.float32) + b2[...], 0.0)
    h = jnp.maximum(jnp.dot(h, w3[...], preferred_element_type=jnp.float32) + b3[...], 0.0)
    z = jnp.dot(h, w4r[...], preferred_element_type=jnp.float32)[:, 0] + b4[0]
    o_ref[...] = 1.0 / (1.0 + jnp.exp(-z))


def _full(shape):
    return pl.BlockSpec(shape, lambda i: tuple(0 for _ in shape))


def _mlp(xg, yg, w1a, w1b, b1, w2, b2, w3, b3, w4r, b4):
    return pl.pallas_call(
        _mlp_body,
        grid=(_B // _BS,),
        in_specs=[
            pl.BlockSpec((_BS, _D), lambda i: (i, 0)),
            pl.BlockSpec((_BS, _D), lambda i: (i, 0)),
            _full((_D, 128)),
            _full((_D, 128)),
            _full((1, 128)),
            _full((128, 64)),
            _full((1, 64)),
            _full((64, 32)),
            _full((1, 32)),
            _full((32, 1)),
            _full((1,)),
        ],
        out_specs=pl.BlockSpec((_BS,), lambda i: (i,)),
        out_shape=jax.ShapeDtypeStruct((_B,), jnp.float32),
        compiler_params=pltpu.CompilerParams(dimension_semantics=("parallel",)),
    )(xg, yg, w1a, w1b, b1, w2, b2, w3, b3, w4r, b4)


def kernel(user_input, item_input, user_emb, W1, b1, W2, b2, W3, b3, W4, b4):
    uidx = user_input.astype(jnp.int32)
    iidx = item_input.astype(jnp.int32)
    xg, yg = _make_gather()(uidx, iidx, user_emb)
    return _mlp(
        xg, yg,
        W1[:_D], W1[_D:],
        b1.reshape(1, -1),
        W2, b2.reshape(1, -1),
        W3, b3.reshape(1, -1),
        W4, b4,
    )
